# Initial kernel scaffold; baseline (speedup 1.0000x reference)
import jax
import jax.numpy as jnp
from jax.experimental import pallas as pl


def kernel(x, Wr, We, be):
    raise NotImplementedError("write your pallas kernel here")


# speedup vs baseline: 1.3929x; 1.3929x over previous
import functools

import jax
import jax.numpy as jnp
from jax import lax
from jax.experimental import pallas as pl
from jax.experimental.pallas import tpu as pltpu
from jax.experimental.pallas import tpu_sc as plsc

N = 16384
D = 2048
E = 8
BM = 256
BN = 256
M_TILES = N // BM
N_TILES = D // BN

_NC, _NS, _L = 2, 16, 16
NW = _NC * _NS
CHUNK = N // NW
ROWS = 32
SUB = CHUNK // ROWS


def _router_body(x_ref, wrt_ref, tri_ref, eid_ref, wgt_ref, rank_ref,
                 cnt_ref, carry_ref):
    @pl.when(pl.program_id(0) == 0)
    def _():
        carry_ref[...] = jnp.zeros_like(carry_ref)

    logits = jax.lax.dot_general(
        x_ref[...], wrt_ref[...], (((1,), (0,)), ((), ())),
        preferred_element_type=jnp.float32)
    lmax = jnp.max(logits, axis=1, keepdims=True)
    ssum = jnp.sum(jnp.exp(logits - lmax), axis=1)
    wgt = 1.0 / ssum
    eid = jnp.argmax(logits, axis=1).astype(jnp.int32)

    onehot = (jax.lax.broadcasted_iota(jnp.int32, (BM, E), 1)
              == eid[:, None]).astype(jnp.float32)
    incl = jax.lax.dot_general(
        tri_ref[...], onehot, (((1,), (0,)), ((), ())),
        preferred_element_type=jnp.float32)
    excl = incl - onehot
    rank_tile = jnp.sum(excl * onehot, axis=1)
    carry = carry_ref[...]
    rank = rank_tile + jnp.sum(carry * onehot, axis=1)

    tile_cnt = jnp.sum(onehot, axis=0, keepdims=True)
    new_carry = carry + tile_cnt
    carry_ref[...] = new_carry
    cnt_ref[...] = new_carry.astype(jnp.int32)

    eid_ref[...] = eid[:, None]
    wgt_ref[...] = wgt[:, None]
    rank_ref[...] = rank[:, None].astype(jnp.int32)


def _router(x, wrt, tri):
    return pl.pallas_call(
        _router_body,
        grid=(M_TILES,),
        in_specs=[
            pl.BlockSpec((BM, D), lambda m: (m, 0)),
            pl.BlockSpec((D, E), lambda m: (0, 0)),
            pl.BlockSpec((BM, BM), lambda m: (0, 0)),
        ],
        out_specs=[
            pl.BlockSpec((BM, 1), lambda m: (m, 0)),
            pl.BlockSpec((BM, 1), lambda m: (m, 0)),
            pl.BlockSpec((BM, 1), lambda m: (m, 0)),
            pl.BlockSpec((1, E), lambda m: (0, 0)),
        ],
        out_shape=[
            jax.ShapeDtypeStruct((N, 1), jnp.int32),
            jax.ShapeDtypeStruct((N, 1), jnp.float32),
            jax.ShapeDtypeStruct((N, 1), jnp.int32),
            jax.ShapeDtypeStruct((1, E), jnp.int32),
        ],
        scratch_shapes=[pltpu.VMEM((1, E), jnp.float32)],
    )(x, wrt, tri)


def _dispatch_body(x_hbm, eid_hbm, rank_hbm, wgt_hbm, offs_hbm,
                   xs_hbm, ws_hbm, src_hbm,
                   eid_v, rank_v, wgt_v, pos_v, offs_v, ids_v, xbuf, sem):
    wid = lax.axis_index("s") * _NC + lax.axis_index("c")
    cb = wid * CHUNK
    pltpu.sync_copy(eid_hbm.at[pl.ds(cb, CHUNK)], eid_v)
    pltpu.sync_copy(rank_hbm.at[pl.ds(cb, CHUNK)], rank_v)
    pltpu.sync_copy(wgt_hbm.at[pl.ds(cb, CHUNK)], wgt_v)
    pltpu.sync_copy(offs_hbm, offs_v)

    lane = lax.broadcasted_iota(jnp.int32, (_L,), 0)
    for j in range(CHUNK // _L):
        sl = pl.ds(j * _L, _L)
        eid16 = eid_v[sl]
        pos16 = plsc.load_gather(offs_v, [eid16]) + rank_v[sl]
        pos_v[sl] = pos16
        ids_v[sl] = cb + j * _L + lane
        pltpu.async_copy(wgt_v.at[sl], ws_hbm.at[pos16], sem).wait()
        pltpu.async_copy(ids_v.at[sl], src_hbm.at[pos16], sem).wait()

    for j in range(SUB):
        pltpu.sync_copy(x_hbm.at[pl.ds(cb + j * ROWS, ROWS)], xbuf)
        for h in range(ROWS // _L):
            pos16 = pos_v[pl.ds(j * ROWS + h * _L, _L)]
            pltpu.async_copy(xbuf.at[pl.ds(h * _L, _L)],
                             xs_hbm.at[pos16], sem).wait()


def _dispatch(x, eid, rank, wgt, offs):
    mesh = plsc.VectorSubcoreMesh(core_axis_name="c", subcore_axis_name="s")
    f = pl.kernel(
        _dispatch_body,
        compiler_params=pltpu.CompilerParams(needs_layout_passes=False),
        out_type=[
            jax.ShapeDtypeStruct((N, D), jnp.float32),
            jax.ShapeDtypeStruct((N,), jnp.float32),
            jax.ShapeDtypeStruct((N,), jnp.int32),
        ],
        mesh=mesh,
        scratch_types=[
            pltpu.VMEM((CHUNK,), jnp.int32),
            pltpu.VMEM((CHUNK,), jnp.int32),
            pltpu.VMEM((CHUNK,), jnp.float32),
            pltpu.VMEM((CHUNK,), jnp.int32),
            pltpu.VMEM((E,), jnp.int32),
            pltpu.VMEM((CHUNK,), jnp.int32),
            pltpu.VMEM((ROWS, D), jnp.float32),
            pltpu.SemaphoreType.DMA,
        ],
    )
    return f(x, eid, rank, wgt, offs)


def _gmm_body(offs_ref, xs_ref, ws_ref, we_ref, be_ref, out_ref):
    m = pl.program_id(1)
    start = m * BM
    rows = start + jax.lax.broadcasted_iota(jnp.int32, (BM, 1), 0)

    e_lo = jnp.int32(0)
    e_hi = jnp.int32(0)
    for e in range(1, E):
        off_e = offs_ref[e]
        e_lo = jnp.where(off_e <= start, jnp.int32(e), e_lo)
        e_hi = jnp.where(off_e <= start + BM - 1, jnp.int32(e), e_hi)

    e_row = jnp.zeros((BM, 1), jnp.int32)
    for e in range(1, E):
        e_row = e_row + (rows >= offs_ref[e]).astype(jnp.int32)
    onehot = (jax.lax.broadcasted_iota(jnp.int32, (BM, E), 1)
              == e_row).astype(jnp.float32)
    acc0 = jax.lax.dot_general(
        onehot, be_ref[...], (((1,), (0,)), ((), ())),
        preferred_element_type=jnp.float32)

    def body(e, acc):
        lo = offs_ref[e]
        hi = offs_ref[e + 1]
        mask = (rows >= lo) & (rows < hi)
        xm = jnp.where(mask, xs_ref[...], 0.0)
        w = we_ref[e]
        return acc + jax.lax.dot_general(
            xm, w, (((1,), (1,)), ((), ())),
            preferred_element_type=jnp.float32)

    acc = jax.lax.fori_loop(e_lo, e_hi + 1, body, acc0)
    out_ref[...] = acc * ws_ref[...]


def _gmm(offs, xs, ws2, We, be):
    grid_spec = pltpu.PrefetchScalarGridSpec(
        num_scalar_prefetch=1,
        grid=(N_TILES, M_TILES),
        in_specs=[
            pl.BlockSpec((BM, D), lambda n, m, offs: (m, 0)),
            pl.BlockSpec((BM, 1), lambda n, m, offs: (m, 0)),
            pl.BlockSpec((E, BN, D), lambda n, m, offs: (0, n, 0)),
            pl.BlockSpec((E, BN), lambda n, m, offs: (0, n)),
        ],
        out_specs=pl.BlockSpec((BM, BN), lambda n, m, offs: (m, n)),
    )
    return pl.pallas_call(
        _gmm_body,
        grid_spec=grid_spec,
        out_shape=jax.ShapeDtypeStruct((N, D), jnp.float32),
    )(offs, xs, ws2, We, be)


def _combine_body(y_hbm, src_hbm, out_hbm, src_v, ybuf, sem):
    wid = lax.axis_index("s") * _NC + lax.axis_index("c")
    cb = wid * CHUNK
    pltpu.sync_copy(src_hbm.at[pl.ds(cb, CHUNK)], src_v)
    for j in range(SUB):
        pltpu.sync_copy(y_hbm.at[pl.ds(cb + j * ROWS, ROWS)], ybuf)
        for h in range(ROWS // _L):
            src16 = src_v[pl.ds(j * ROWS + h * _L, _L)]
            pltpu.async_copy(ybuf.at[pl.ds(h * _L, _L)],
                             out_hbm.at[src16], sem).wait()


def _combine(y, src):
    mesh = plsc.VectorSubcoreMesh(core_axis_name="c", subcore_axis_name="s")
    f = pl.kernel(
        _combine_body,
        out_type=jax.ShapeDtypeStruct((N, D), jnp.float32),
        mesh=mesh,
        scratch_types=[
            pltpu.VMEM((CHUNK,), jnp.int32),
            pltpu.VMEM((ROWS, D), jnp.float32),
            pltpu.SemaphoreType.DMA,
        ],
    )
    return f(y, src)


@jax.jit
def kernel(x, Wr, We, be):
    tri = jnp.tril(jnp.ones((BM, BM), jnp.float32))
    eid2, wgt2, rank2, cnt = _router(x, Wr.T, tri)
    counts = cnt[0]
    offs9 = jnp.concatenate(
        [jnp.zeros((1,), jnp.int32), jnp.cumsum(counts).astype(jnp.int32)])
    xs, ws, src = _dispatch(x, eid2[:, 0], rank2[:, 0], wgt2[:, 0], offs9[:E])
    y = _gmm(offs9, xs, ws[:, None], We, be)
    return _combine(y, src)


# speedup vs baseline: 2.5227x; 1.8112x over previous
import functools

import jax
import jax.numpy as jnp
from jax import lax
from jax.experimental import pallas as pl
from jax.experimental.pallas import tpu as pltpu
from jax.experimental.pallas import tpu_sc as plsc

N = 16384
D = 2048
E = 8
BM = 256
M_TILES = N // BM

_NC, _NS, _L = 2, 16, 16
NW = _NC * _NS
CHUNK = N // NW
ROWS = 32
SUB = CHUNK // ROWS


def _router_body(x_ref, wrt_ref, tri_ref, eid_ref, wgt_ref, rank_ref,
                 cnt_ref, carry_ref):
    @pl.when(pl.program_id(0) == 0)
    def _():
        carry_ref[...] = jnp.zeros_like(carry_ref)

    logits = jax.lax.dot_general(
        x_ref[...], wrt_ref[...], (((1,), (0,)), ((), ())),
        preferred_element_type=jnp.float32)
    lmax = jnp.max(logits, axis=1, keepdims=True)
    ssum = jnp.sum(jnp.exp(logits - lmax), axis=1)
    wgt = 1.0 / ssum
    eid = jnp.argmax(logits, axis=1).astype(jnp.int32)

    onehot = (jax.lax.broadcasted_iota(jnp.int32, (BM, E), 1)
              == eid[:, None]).astype(jnp.float32)
    incl = jax.lax.dot_general(
        tri_ref[...], onehot, (((1,), (0,)), ((), ())),
        preferred_element_type=jnp.float32)
    excl = incl - onehot
    rank_tile = jnp.sum(excl * onehot, axis=1)
    carry = carry_ref[...]
    rank = rank_tile + jnp.sum(carry * onehot, axis=1)

    tile_cnt = jnp.sum(onehot, axis=0, keepdims=True)
    new_carry = carry + tile_cnt
    carry_ref[...] = new_carry
    cnt_ref[...] = new_carry.astype(jnp.int32)

    eid_ref[...] = eid[:, None]
    wgt_ref[...] = wgt[:, None]
    rank_ref[...] = rank[:, None].astype(jnp.int32)


def _router(x, wrt, tri):
    return pl.pallas_call(
        _router_body,
        grid=(M_TILES,),
        in_specs=[
            pl.BlockSpec((BM, D), lambda m: (m, 0)),
            pl.BlockSpec((D, E), lambda m: (0, 0)),
            pl.BlockSpec((BM, BM), lambda m: (0, 0)),
        ],
        out_specs=[
            pl.BlockSpec((BM, 1), lambda m: (m, 0)),
            pl.BlockSpec((BM, 1), lambda m: (m, 0)),
            pl.BlockSpec((BM, 1), lambda m: (m, 0)),
            pl.BlockSpec((1, E), lambda m: (0, 0)),
        ],
        out_shape=[
            jax.ShapeDtypeStruct((N, 1), jnp.int32),
            jax.ShapeDtypeStruct((N, 1), jnp.float32),
            jax.ShapeDtypeStruct((N, 1), jnp.int32),
            jax.ShapeDtypeStruct((1, E), jnp.int32),
        ],
        scratch_shapes=[pltpu.VMEM((1, E), jnp.float32)],
    )(x, wrt, tri)


def _dispatch_body(x_hbm, eid_hbm, rank_hbm, wgt_hbm, offs_hbm,
                   xs_hbm, ws_hbm, src_hbm,
                   eid_v, rank_v, wgt_v, pos_v, offs_v, ids_v, xbuf, sem):
    wid = lax.axis_index("s") * _NC + lax.axis_index("c")
    cb = wid * CHUNK
    pltpu.sync_copy(eid_hbm.at[pl.ds(cb, CHUNK)], eid_v)
    pltpu.sync_copy(rank_hbm.at[pl.ds(cb, CHUNK)], rank_v)
    pltpu.sync_copy(wgt_hbm.at[pl.ds(cb, CHUNK)], wgt_v)
    pltpu.sync_copy(offs_hbm, offs_v)

    lane = lax.broadcasted_iota(jnp.int32, (_L,), 0)
    for j in range(CHUNK // _L):
        sl = pl.ds(j * _L, _L)
        eid16 = eid_v[sl]
        pos16 = plsc.load_gather(offs_v, [eid16]) + rank_v[sl]
        pos_v[sl] = pos16
        ids_v[sl] = cb + j * _L + lane
        pltpu.async_copy(wgt_v.at[sl], ws_hbm.at[pos16], sem).wait()
        pltpu.async_copy(ids_v.at[sl], src_hbm.at[pos16], sem).wait()

    for j in range(SUB):
        pltpu.sync_copy(x_hbm.at[pl.ds(cb + j * ROWS, ROWS)], xbuf)
        for h in range(ROWS // _L):
            pos16 = pos_v[pl.ds(j * ROWS + h * _L, _L)]
            pltpu.async_copy(xbuf.at[pl.ds(h * _L, _L)],
                             xs_hbm.at[pos16], sem).wait()


def _dispatch(x, eid, rank, wgt, offs):
    mesh = plsc.VectorSubcoreMesh(core_axis_name="c", subcore_axis_name="s")
    f = pl.kernel(
        _dispatch_body,
        compiler_params=pltpu.CompilerParams(needs_layout_passes=False),
        out_type=[
            jax.ShapeDtypeStruct((N, D), jnp.float32),
            jax.ShapeDtypeStruct((N,), jnp.float32),
            jax.ShapeDtypeStruct((N,), jnp.int32),
        ],
        mesh=mesh,
        scratch_types=[
            pltpu.VMEM((CHUNK,), jnp.int32),
            pltpu.VMEM((CHUNK,), jnp.int32),
            pltpu.VMEM((CHUNK,), jnp.float32),
            pltpu.VMEM((CHUNK,), jnp.int32),
            pltpu.VMEM((E,), jnp.int32),
            pltpu.VMEM((CHUNK,), jnp.int32),
            pltpu.VMEM((ROWS, D), jnp.float32),
            pltpu.SemaphoreType.DMA,
        ],
    )
    return f(x, eid, rank, wgt, offs)


ITEMS = M_TILES + E - 1


def _gmm_body(offs_ref, m_idx_ref, e_idx_ref, lo_ref, hi_ref, first_ref,
              last_ref, xs_ref, ws_ref, we_ref, be_ref, out_ref):
    i = pl.program_id(0)
    m = m_idx_ref[i]
    rows = m * BM + jax.lax.broadcasted_iota(jnp.int32, (BM, 1), 0)
    mask = (rows >= lo_ref[i]) & (rows < hi_ref[i])
    xm = jnp.where(mask, xs_ref[...], 0.0)
    contrib = jax.lax.dot_general(
        xm, we_ref[0], (((1,), (1,)), ((), ())),
        preferred_element_type=jnp.float32)

    e_row = jnp.zeros((BM, 1), jnp.int32)
    for e in range(1, E):
        e_row = e_row + (rows >= offs_ref[e]).astype(jnp.int32)
    onehot = (jax.lax.broadcasted_iota(jnp.int32, (BM, E), 1)
              == e_row).astype(jnp.float32)
    bias = jax.lax.dot_general(
        onehot, be_ref[...], (((1,), (0,)), ((), ())),
        preferred_element_type=jnp.float32)

    prev = jnp.where(first_ref[i] == 1, bias, out_ref[...])
    tmp = prev + contrib
    out_ref[...] = jnp.where(last_ref[i] == 1, tmp * ws_ref[...], tmp)


def _gmm(offs9, xs, ws2, We, be):
    starts = jnp.arange(M_TILES, dtype=jnp.int32) * BM
    inner = offs9[1:E]
    e_lo_t = jnp.searchsorted(inner, starts, side="right").astype(jnp.int32)
    e_hi_t = jnp.searchsorted(inner, starts + BM - 1,
                              side="right").astype(jnp.int32)
    span = e_hi_t - e_lo_t + 1
    item_start = jnp.cumsum(span) - span
    total = item_start[-1] + span[-1]
    ii = jnp.arange(ITEMS, dtype=jnp.int32)
    t_i = (jnp.searchsorted(item_start, ii, side="right") - 1).astype(jnp.int32)
    valid = ii < total
    t_i = jnp.where(valid, t_i, M_TILES - 1)
    e_i = jnp.where(valid, e_lo_t[t_i] + ii - item_start[t_i], 0)
    lo_a = jnp.where(valid, offs9[e_i], 0)
    hi_a = jnp.where(valid, offs9[e_i + 1], 0)
    first_a = (valid & (ii == item_start[t_i])).astype(jnp.int32)
    last_a = (valid & (ii == item_start[t_i] + span[t_i] - 1)).astype(jnp.int32)

    grid_spec = pltpu.PrefetchScalarGridSpec(
        num_scalar_prefetch=7,
        grid=(ITEMS,),
        in_specs=[
            pl.BlockSpec((BM, D), lambda i, o, mi, ei, lo, hi, f, l: (mi[i], 0)),
            pl.BlockSpec((BM, 1), lambda i, o, mi, ei, lo, hi, f, l: (mi[i], 0)),
            pl.BlockSpec((1, D, D), lambda i, o, mi, ei, lo, hi, f, l: (ei[i], 0, 0)),
            pl.BlockSpec((E, D), lambda i, o, mi, ei, lo, hi, f, l: (0, 0)),
        ],
        out_specs=pl.BlockSpec((BM, D), lambda i, o, mi, ei, lo, hi, f, l: (mi[i], 0)),
    )
    return pl.pallas_call(
        _gmm_body,
        grid_spec=grid_spec,
        out_shape=jax.ShapeDtypeStruct((N, D), jnp.float32),
    )(offs9, t_i, e_i, lo_a, hi_a, first_a, last_a, xs, ws2, We, be)


def _combine_body(y_hbm, src_hbm, out_hbm, src_v, ybuf, sem):
    wid = lax.axis_index("s") * _NC + lax.axis_index("c")
    cb = wid * CHUNK
    pltpu.sync_copy(src_hbm.at[pl.ds(cb, CHUNK)], src_v)
    for j in range(SUB):
        pltpu.sync_copy(y_hbm.at[pl.ds(cb + j * ROWS, ROWS)], ybuf)
        for h in range(ROWS // _L):
            src16 = src_v[pl.ds(j * ROWS + h * _L, _L)]
            pltpu.async_copy(ybuf.at[pl.ds(h * _L, _L)],
                             out_hbm.at[src16], sem).wait()


def _combine(y, src):
    mesh = plsc.VectorSubcoreMesh(core_axis_name="c", subcore_axis_name="s")
    f = pl.kernel(
        _combine_body,
        out_type=jax.ShapeDtypeStruct((N, D), jnp.float32),
        mesh=mesh,
        scratch_types=[
            pltpu.VMEM((CHUNK,), jnp.int32),
            pltpu.VMEM((ROWS, D), jnp.float32),
            pltpu.SemaphoreType.DMA,
        ],
    )
    return f(y, src)


@jax.jit
def kernel(x, Wr, We, be):
    tri = jnp.tril(jnp.ones((BM, BM), jnp.float32))
    eid2, wgt2, rank2, cnt = _router(x, Wr.T, tri)
    counts = cnt[0]
    offs9 = jnp.concatenate(
        [jnp.zeros((1,), jnp.int32), jnp.cumsum(counts).astype(jnp.int32)])
    xs, ws, src = _dispatch(x, eid2[:, 0], rank2[:, 0], wgt2[:, 0], offs9[:E])
    y = _gmm(offs9, xs, ws[:, None], We, be)
    return _combine(y, src)


# speedup vs baseline: 2.5953x; 1.0288x over previous
import functools

import jax
import jax.numpy as jnp
from jax import lax
from jax.experimental import pallas as pl
from jax.experimental.pallas import tpu as pltpu
from jax.experimental.pallas import tpu_sc as plsc

N = 16384
D = 2048
E = 8
BM = 256
M_TILES = N // BM

_NC, _NS, _L = 2, 16, 16
NW = _NC * _NS
CHUNK = N // NW
ROWS = 16
SUB = CHUNK // ROWS
NBUF = 3


def _router_body(x_ref, wrt_ref, tri_ref, eid_ref, wgt_ref, rank_ref,
                 cnt_ref, carry_ref):
    @pl.when(pl.program_id(0) == 0)
    def _():
        carry_ref[...] = jnp.zeros_like(carry_ref)

    logits = jax.lax.dot_general(
        x_ref[...], wrt_ref[...], (((1,), (0,)), ((), ())),
        preferred_element_type=jnp.float32)
    lmax = jnp.max(logits, axis=1, keepdims=True)
    ssum = jnp.sum(jnp.exp(logits - lmax), axis=1)
    wgt = 1.0 / ssum
    eid = jnp.argmax(logits, axis=1).astype(jnp.int32)

    onehot = (jax.lax.broadcasted_iota(jnp.int32, (BM, E), 1)
              == eid[:, None]).astype(jnp.float32)
    incl = jax.lax.dot_general(
        tri_ref[...], onehot, (((1,), (0,)), ((), ())),
        preferred_element_type=jnp.float32)
    excl = incl - onehot
    rank_tile = jnp.sum(excl * onehot, axis=1)
    carry = carry_ref[...]
    rank = rank_tile + jnp.sum(carry * onehot, axis=1)

    tile_cnt = jnp.sum(onehot, axis=0, keepdims=True)
    new_carry = carry + tile_cnt
    carry_ref[...] = new_carry
    cnt_ref[...] = new_carry.astype(jnp.int32)

    eid_ref[...] = eid[:, None]
    wgt_ref[...] = wgt[:, None]
    rank_ref[...] = rank[:, None].astype(jnp.int32)


def _router(x, wrt, tri):
    return pl.pallas_call(
        _router_body,
        grid=(M_TILES,),
        in_specs=[
            pl.BlockSpec((BM, D), lambda m: (m, 0)),
            pl.BlockSpec((D, E), lambda m: (0, 0)),
            pl.BlockSpec((BM, BM), lambda m: (0, 0)),
        ],
        out_specs=[
            pl.BlockSpec((BM, 1), lambda m: (m, 0)),
            pl.BlockSpec((BM, 1), lambda m: (m, 0)),
            pl.BlockSpec((BM, 1), lambda m: (m, 0)),
            pl.BlockSpec((1, E), lambda m: (0, 0)),
        ],
        out_shape=[
            jax.ShapeDtypeStruct((N, 1), jnp.int32),
            jax.ShapeDtypeStruct((N, 1), jnp.float32),
            jax.ShapeDtypeStruct((N, 1), jnp.int32),
            jax.ShapeDtypeStruct((1, E), jnp.int32),
        ],
        scratch_shapes=[pltpu.VMEM((1, E), jnp.float32)],
    )(x, wrt, tri)


def _dispatch_body(x_hbm, eid_hbm, rank_hbm, wgt_hbm, offs_hbm,
                   xs_hbm, ws_hbm, src_hbm,
                   eid_v, rank_v, wgt_v, pos_m, offs_v, ids_v,
                   b0, b1, b2, sg0, sg1, sg2, ss0, ss1, ss2, sca):
    wid = lax.axis_index("s") * _NC + lax.axis_index("c")
    cb = wid * CHUNK
    bufs, sg, ss = [b0, b1, b2], [sg0, sg1, sg2], [ss0, ss1, ss2]

    gcp = {}
    for j in range(NBUF):
        gcp[j] = pltpu.async_copy(
            x_hbm.at[pl.ds(cb + j * ROWS, ROWS)], bufs[j], sg[j])
    pltpu.sync_copy(eid_hbm.at[pl.ds(cb, CHUNK)], eid_v)
    pltpu.sync_copy(rank_hbm.at[pl.ds(cb, CHUNK)], rank_v)
    pltpu.sync_copy(wgt_hbm.at[pl.ds(cb, CHUNK)], wgt_v)
    pltpu.sync_copy(offs_hbm, offs_v)

    lane = lax.broadcasted_iota(jnp.int32, (_L,), 0)
    for j in range(CHUNK // _L):
        sl = pl.ds(j * _L, _L)
        pos16 = plsc.load_gather(offs_v, [eid_v[sl]]) + rank_v[sl]
        pos_m[j // 8, pl.ds((j % 8) * _L, _L)] = pos16
        ids_v[sl] = cb + j * _L + lane

    scp = []
    for r in range(CHUNK // 128):
        scp.append(pltpu.async_copy(
            wgt_v.at[pl.ds(r * 128, 128)], ws_hbm.at[pos_m.at[r]], sca))
        scp.append(pltpu.async_copy(
            ids_v.at[pl.ds(r * 128, 128)], src_hbm.at[pos_m.at[r]], sca))

    for j in range(SUB):
        b = j % NBUF
        gcp[j].wait()
        pos16 = pos_m[j // 8, pl.ds((j % 8) * _L, _L)]
        sc = pltpu.async_copy(bufs[b], xs_hbm.at[pos16], ss[b])
        if j + NBUF < SUB:
            sc.wait()
            gcp[j + NBUF] = pltpu.async_copy(
                x_hbm.at[pl.ds(cb + (j + NBUF) * ROWS, ROWS)], bufs[b], sg[b])
        else:
            scp.append(sc)
    for c in scp:
        c.wait()


def _dispatch(x, eid, rank, wgt, offs):
    mesh = plsc.VectorSubcoreMesh(core_axis_name="c", subcore_axis_name="s")
    f = pl.kernel(
        _dispatch_body,
        compiler_params=pltpu.CompilerParams(needs_layout_passes=False),
        out_type=[
            jax.ShapeDtypeStruct((N, D), jnp.float32),
            jax.ShapeDtypeStruct((N,), jnp.float32),
            jax.ShapeDtypeStruct((N,), jnp.int32),
        ],
        mesh=mesh,
        scratch_types=[
            pltpu.VMEM((CHUNK,), jnp.int32),
            pltpu.VMEM((CHUNK,), jnp.int32),
            pltpu.VMEM((CHUNK,), jnp.float32),
            pltpu.VMEM((CHUNK // 128, 128), jnp.int32),
            pltpu.VMEM((E,), jnp.int32),
            pltpu.VMEM((CHUNK,), jnp.int32),
            pltpu.VMEM((ROWS, D), jnp.float32),
            pltpu.VMEM((ROWS, D), jnp.float32),
            pltpu.VMEM((ROWS, D), jnp.float32),
            pltpu.SemaphoreType.DMA,
            pltpu.SemaphoreType.DMA,
            pltpu.SemaphoreType.DMA,
            pltpu.SemaphoreType.DMA,
            pltpu.SemaphoreType.DMA,
            pltpu.SemaphoreType.DMA,
            pltpu.SemaphoreType.DMA,
        ],
    )
    return f(x, eid, rank, wgt, offs)


ITEMS = M_TILES + E - 1


def _gmm_body(offs_ref, m_idx_ref, e_idx_ref, lo_ref, hi_ref, first_ref,
              last_ref, xs_ref, ws_ref, we_ref, be_ref, out_ref):
    i = pl.program_id(0)
    m = m_idx_ref[i]
    rows = m * BM + jax.lax.broadcasted_iota(jnp.int32, (BM, 1), 0)
    mask = (rows >= lo_ref[i]) & (rows < hi_ref[i])
    xm = jnp.where(mask, xs_ref[...], 0.0)
    contrib = jax.lax.dot_general(
        xm, we_ref[0], (((1,), (1,)), ((), ())),
        preferred_element_type=jnp.float32)

    e_row = jnp.zeros((BM, 1), jnp.int32)
    for e in range(1, E):
        e_row = e_row + (rows >= offs_ref[e]).astype(jnp.int32)
    onehot = (jax.lax.broadcasted_iota(jnp.int32, (BM, E), 1)
              == e_row).astype(jnp.float32)
    bias = jax.lax.dot_general(
        onehot, be_ref[...], (((1,), (0,)), ((), ())),
        preferred_element_type=jnp.float32)

    prev = jnp.where(first_ref[i] == 1, bias, out_ref[...])
    tmp = prev + contrib
    out_ref[...] = jnp.where(last_ref[i] == 1, tmp * ws_ref[...], tmp)


def _gmm(offs9, xs, ws2, We, be):
    starts = jnp.arange(M_TILES, dtype=jnp.int32) * BM
    inner = offs9[1:E]
    e_lo_t = jnp.searchsorted(inner, starts, side="right").astype(jnp.int32)
    e_hi_t = jnp.searchsorted(inner, starts + BM - 1,
                              side="right").astype(jnp.int32)
    span = e_hi_t - e_lo_t + 1
    item_start = jnp.cumsum(span) - span
    total = item_start[-1] + span[-1]
    ii = jnp.arange(ITEMS, dtype=jnp.int32)
    t_i = (jnp.searchsorted(item_start, ii, side="right") - 1).astype(jnp.int32)
    valid = ii < total
    t_i = jnp.where(valid, t_i, M_TILES - 1)
    e_i = jnp.where(valid, e_lo_t[t_i] + ii - item_start[t_i], 0)
    lo_a = jnp.where(valid, offs9[e_i], 0)
    hi_a = jnp.where(valid, offs9[e_i + 1], 0)
    first_a = (valid & (ii == item_start[t_i])).astype(jnp.int32)
    last_a = (valid & (ii == item_start[t_i] + span[t_i] - 1)).astype(jnp.int32)

    grid_spec = pltpu.PrefetchScalarGridSpec(
        num_scalar_prefetch=7,
        grid=(ITEMS,),
        in_specs=[
            pl.BlockSpec((BM, D), lambda i, o, mi, ei, lo, hi, f, l: (mi[i], 0)),
            pl.BlockSpec((BM, 1), lambda i, o, mi, ei, lo, hi, f, l: (mi[i], 0)),
            pl.BlockSpec((1, D, D), lambda i, o, mi, ei, lo, hi, f, l: (ei[i], 0, 0)),
            pl.BlockSpec((E, D), lambda i, o, mi, ei, lo, hi, f, l: (0, 0)),
        ],
        out_specs=pl.BlockSpec((BM, D), lambda i, o, mi, ei, lo, hi, f, l: (mi[i], 0)),
    )
    return pl.pallas_call(
        _gmm_body,
        grid_spec=grid_spec,
        out_shape=jax.ShapeDtypeStruct((N, D), jnp.float32),
    )(offs9, t_i, e_i, lo_a, hi_a, first_a, last_a, xs, ws2, We, be)


def _combine_body(y_hbm, src_hbm, out_hbm, src_v,
                  b0, b1, b2, sg0, sg1, sg2, ss0, ss1, ss2):
    wid = lax.axis_index("s") * _NC + lax.axis_index("c")
    cb = wid * CHUNK
    bufs, sg, ss = [b0, b1, b2], [sg0, sg1, sg2], [ss0, ss1, ss2]
    gcp = {}
    for j in range(NBUF):
        gcp[j] = pltpu.async_copy(
            y_hbm.at[pl.ds(cb + j * ROWS, ROWS)], bufs[j], sg[j])
    pltpu.sync_copy(src_hbm.at[pl.ds(cb, CHUNK)], src_v)
    tail = []
    for j in range(SUB):
        b = j % NBUF
        gcp[j].wait()
        src16 = src_v[pl.ds(j * ROWS, _L)]
        sc = pltpu.async_copy(bufs[b], out_hbm.at[src16], ss[b])
        if j + NBUF < SUB:
            sc.wait()
            gcp[j + NBUF] = pltpu.async_copy(
                y_hbm.at[pl.ds(cb + (j + NBUF) * ROWS, ROWS)], bufs[b], sg[b])
        else:
            tail.append(sc)
    for c in tail:
        c.wait()


def _combine(y, src):
    mesh = plsc.VectorSubcoreMesh(core_axis_name="c", subcore_axis_name="s")
    f = pl.kernel(
        _combine_body,
        out_type=jax.ShapeDtypeStruct((N, D), jnp.float32),
        mesh=mesh,
        scratch_types=[
            pltpu.VMEM((CHUNK,), jnp.int32),
            pltpu.VMEM((ROWS, D), jnp.float32),
            pltpu.VMEM((ROWS, D), jnp.float32),
            pltpu.VMEM((ROWS, D), jnp.float32),
            pltpu.SemaphoreType.DMA,
            pltpu.SemaphoreType.DMA,
            pltpu.SemaphoreType.DMA,
            pltpu.SemaphoreType.DMA,
            pltpu.SemaphoreType.DMA,
            pltpu.SemaphoreType.DMA,
        ],
    )
    return f(y, src)


@jax.jit
def kernel(x, Wr, We, be):
    tri = jnp.tril(jnp.ones((BM, BM), jnp.float32))
    eid2, wgt2, rank2, cnt = _router(x, Wr.T, tri)
    counts = cnt[0]
    offs9 = jnp.concatenate(
        [jnp.zeros((1,), jnp.int32), jnp.cumsum(counts).astype(jnp.int32)])
    xs, ws, src = _dispatch(x, eid2[:, 0], rank2[:, 0], wgt2[:, 0], offs9[:E])
    y = _gmm(offs9, xs, ws[:, None], We, be)
    return _combine(y, src)
